# baseline (speedup 1.0000x reference)
import jax
import jax.numpy as jnp
from jax.experimental import pallas as pl
from jax.experimental.pallas import tpu as pltpu

T = 32768
H = 768
E = 64
TOP_K = 2
CH = 4096
NCH = T // CH
NBUF = 3


def _compute_chunk(x, w, b):
    logits = jax.lax.dot_general(
        x, w, (((1,), (0,)), ((), ())),
        preferred_element_type=jnp.float32) + b
    m = jnp.max(logits, axis=1, keepdims=True)
    ex = jnp.exp(logits - m)
    aff = ex / jnp.sum(ex, axis=1, keepdims=True)
    iota = jax.lax.broadcasted_iota(jnp.int32, (CH, E), 1)
    big = jnp.int32(E)
    top1 = jnp.max(aff, axis=1, keepdims=True)
    idx1 = jnp.min(jnp.where(aff == top1, iota, big), axis=1, keepdims=True)
    masked = jnp.where(iota == idx1, -jnp.inf, aff)
    top2 = jnp.max(masked, axis=1, keepdims=True)
    idx2 = jnp.min(jnp.where(masked == top2, iota, big), axis=1, keepdims=True)
    idx = jnp.concatenate([idx1, idx2], axis=1)
    return logits, aff, idx


def _router(x_hbm, w_hbm, b_hbm, logits_hbm, aff_hbm, idx_hbm,
            xb, wb, bb, lg, af, ix, in_sems, out_sems, w_sem):
    cw = pltpu.make_async_copy(w_hbm, wb, w_sem)
    cw.start()
    cb = pltpu.make_async_copy(b_hbm, bb, w_sem)
    cb.start()
    for j in range(min(NBUF, NCH)):
        pltpu.make_async_copy(
            x_hbm.at[pl.ds(j * CH, CH)], xb.at[j], in_sems.at[j]).start()
    cw.wait()
    cb.wait()

    for i in range(NCH):
        sl = i % NBUF
        pltpu.make_async_copy(
            x_hbm.at[pl.ds(i * CH, CH)], xb.at[sl], in_sems.at[sl]).wait()

        if i >= NBUF:
            prev = (i - NBUF) * CH
            pltpu.make_async_copy(
                lg.at[sl], logits_hbm.at[pl.ds(prev, CH)], out_sems.at[sl]).wait()
            pltpu.make_async_copy(
                af.at[sl], aff_hbm.at[pl.ds(prev, CH)], out_sems.at[sl]).wait()
            pltpu.make_async_copy(
                ix.at[sl], idx_hbm.at[pl.ds(prev, CH)], out_sems.at[sl]).wait()

        logits, aff, idx = _compute_chunk(xb[sl], wb[...], bb[...])
        lg[sl] = logits
        af[sl] = aff
        ix[sl] = idx

        nxt = i + NBUF
        if nxt < NCH:
            pltpu.make_async_copy(
                x_hbm.at[pl.ds(nxt * CH, CH)], xb.at[sl],
                in_sems.at[sl]).start()

        base = i * CH
        pltpu.make_async_copy(
            lg.at[sl], logits_hbm.at[pl.ds(base, CH)], out_sems.at[sl]).start()
        pltpu.make_async_copy(
            af.at[sl], aff_hbm.at[pl.ds(base, CH)], out_sems.at[sl]).start()
        pltpu.make_async_copy(
            ix.at[sl], idx_hbm.at[pl.ds(base, CH)], out_sems.at[sl]).start()

    for i in range(max(NCH - NBUF, 0), NCH):
        sl = i % NBUF
        base = i * CH
        pltpu.make_async_copy(
            lg.at[sl], logits_hbm.at[pl.ds(base, CH)], out_sems.at[sl]).wait()
        pltpu.make_async_copy(
            af.at[sl], aff_hbm.at[pl.ds(base, CH)], out_sems.at[sl]).wait()
        pltpu.make_async_copy(
            ix.at[sl], idx_hbm.at[pl.ds(base, CH)], out_sems.at[sl]).wait()


def kernel(hidden_states, W, b):
    b2 = b.reshape(1, E)
    logits, aff, idx = pl.pallas_call(
        _router,
        in_specs=[
            pl.BlockSpec(memory_space=pl.ANY),
            pl.BlockSpec(memory_space=pl.ANY),
            pl.BlockSpec(memory_space=pl.ANY),
        ],
        out_specs=[
            pl.BlockSpec(memory_space=pl.ANY),
            pl.BlockSpec(memory_space=pl.ANY),
            pl.BlockSpec(memory_space=pl.ANY),
        ],
        out_shape=[
            jax.ShapeDtypeStruct((T, E), jnp.float32),
            jax.ShapeDtypeStruct((T, E), jnp.float32),
            jax.ShapeDtypeStruct((T, TOP_K), jnp.int32),
        ],
        scratch_shapes=[
            pltpu.VMEM((NBUF, CH, H), jnp.float32),
            pltpu.VMEM((H, E), jnp.float32),
            pltpu.VMEM((1, E), jnp.float32),
            pltpu.VMEM((NBUF, CH, E), jnp.float32),
            pltpu.VMEM((NBUF, CH, E), jnp.float32),
            pltpu.VMEM((NBUF, CH, TOP_K), jnp.int32),
            pltpu.SemaphoreType.DMA((NBUF,)),
            pltpu.SemaphoreType.DMA((NBUF,)),
            pltpu.SemaphoreType.DMA,
        ],
        compiler_params=pltpu.CompilerParams(
            vmem_limit_bytes=100 * 1024 * 1024,
        ),
    )(hidden_states, W, b2)
    return (logits, aff, idx)


# speedup vs baseline: 1.0057x; 1.0057x over previous
import functools

import jax
import jax.numpy as jnp
from jax import lax
from jax.experimental import pallas as pl
from jax.experimental.pallas import tpu as pltpu
from jax.experimental.pallas import tpu_sc as plsc

T = 32768
H = 768
E = 64
TOP_K = 2
BT = 4096

NC = 2
NS = 16
L = 16
NW = NC * NS
TPW = T // NW


def _router_block(x_ref, w_ref, b_ref, logits_ref, aff_ref):
    x = x_ref[...]
    w = w_ref[...]
    b = b_ref[...]
    logits = jax.lax.dot_general(
        x, w, (((1,), (0,)), ((), ())),
        preferred_element_type=jnp.float32) + b
    logits_ref[...] = logits
    m = jnp.max(logits, axis=1, keepdims=True)
    ex = jnp.exp(logits - m)
    aff_ref[...] = ex / jnp.sum(ex, axis=1, keepdims=True)


def _tc_logits_softmax(hidden_states, W, b2):
    grid = (T // BT,)
    return pl.pallas_call(
        _router_block,
        grid=grid,
        in_specs=[
            pl.BlockSpec((BT, H), lambda i: (i, 0)),
            pl.BlockSpec((H, E), lambda i: (0, 0)),
            pl.BlockSpec((1, E), lambda i: (0, 0)),
        ],
        out_specs=[
            pl.BlockSpec((BT, E), lambda i: (i, 0)),
            pl.BlockSpec((BT, E), lambda i: (i, 0)),
        ],
        out_shape=[
            jax.ShapeDtypeStruct((T, E), jnp.float32),
            jax.ShapeDtypeStruct((T, E), jnp.float32),
        ],
    )(hidden_states, W, b2)


CHW = 256


def _sc_top2_body(aff_hbm, out1_hbm, out2_hbm, aff_v, i1_v, i2_v):
    wid = lax.axis_index("s") * NC + lax.axis_index("c")
    base = wid * TPW

    lanes = lax.broadcasted_iota(jnp.int32, (L,), 0)

    def group(g, carry):
        tok = lanes + g * L
        t1 = jnp.full((L,), -1.0, jnp.float32)
        t2 = jnp.full((L,), -1.0, jnp.float32)
        i1 = jnp.zeros((L,), jnp.int32)
        i2 = jnp.zeros((L,), jnp.int32)
        for e in range(E):
            ev = jnp.full((L,), e, jnp.int32)
            v = plsc.load_gather(aff_v, [tok, ev])
            gt1 = v > t1
            gt2 = v > t2
            t2 = jnp.where(gt1, t1, jnp.where(gt2, v, t2))
            i2 = jnp.where(gt1, i1, jnp.where(gt2, ev, i2))
            t1 = jnp.where(gt1, v, t1)
            i1 = jnp.where(gt1, ev, i1)
        i1_v[pl.ds(g * L, L)] = i1
        i2_v[pl.ds(g * L, L)] = i2
        return carry

    for c in range(TPW // CHW):
        cbase = base + c * CHW
        pltpu.sync_copy(aff_hbm.at[pl.ds(cbase, CHW)], aff_v)
        lax.fori_loop(0, CHW // L, group, 0)
        pltpu.sync_copy(i1_v, out1_hbm.at[pl.ds(cbase, CHW)])
        pltpu.sync_copy(i2_v, out2_hbm.at[pl.ds(cbase, CHW)])


@functools.partial(
    pl.kernel,
    out_type=[
        jax.ShapeDtypeStruct((T,), jnp.int32),
        jax.ShapeDtypeStruct((T,), jnp.int32),
    ],
    mesh=plsc.VectorSubcoreMesh(core_axis_name="c", subcore_axis_name="s"),
    scratch_types=[
        pltpu.VMEM((CHW, E), jnp.float32),
        pltpu.VMEM((CHW,), jnp.int32),
        pltpu.VMEM((CHW,), jnp.int32),
    ],
    compiler_params=pltpu.CompilerParams(needs_layout_passes=False),
)
def _sc_top2(aff_hbm, out1_hbm, out2_hbm, aff_v, i1_v, i2_v):
    _sc_top2_body(aff_hbm, out1_hbm, out2_hbm, aff_v, i1_v, i2_v)


def kernel(hidden_states, W, b):
    b2 = b.reshape(1, E)
    logits, aff = _tc_logits_softmax(hidden_states, W, b2)
    i1, i2 = _sc_top2(aff)
    idx = jnp.concatenate([i1[:, None], i2[:, None]], axis=1)
    return (logits, aff, idx)


# speedup vs baseline: 1.1179x; 1.1115x over previous
import functools

import jax
import jax.numpy as jnp
from jax import lax
from jax.experimental import pallas as pl
from jax.experimental.pallas import tpu as pltpu
from jax.experimental.pallas import tpu_sc as plsc

T = 32768
H = 768
E = 64
TOP_K = 2
BT = 4096

NC = 2
NS = 16
L = 16
NW = NC * NS
TPW = T // NW


def _router_block(x_ref, w_ref, b_ref, logits_ref, aff_ref):
    x = x_ref[...]
    w = w_ref[...]
    b = b_ref[...]
    logits = jax.lax.dot_general(
        x, w, (((1,), (0,)), ((), ())),
        preferred_element_type=jnp.float32) + b
    logits_ref[...] = logits
    m = jnp.max(logits, axis=1, keepdims=True)
    ex = jnp.exp(logits - m)
    aff_ref[...] = ex / jnp.sum(ex, axis=1, keepdims=True)


def _tc_logits_softmax(hidden_states, W, b2):
    grid = (T // BT,)
    return pl.pallas_call(
        _router_block,
        grid=grid,
        in_specs=[
            pl.BlockSpec((BT, H), lambda i: (i, 0)),
            pl.BlockSpec((H, E), lambda i: (0, 0)),
            pl.BlockSpec((1, E), lambda i: (0, 0)),
        ],
        out_specs=[
            pl.BlockSpec((BT, E), lambda i: (i, 0)),
            pl.BlockSpec((BT, E), lambda i: (i, 0)),
        ],
        out_shape=[
            jax.ShapeDtypeStruct((T, E), jnp.float32),
            jax.ShapeDtypeStruct((T, E), jnp.float32),
        ],
    )(hidden_states, W, b2)


CHW = 256


NCHK = TPW // CHW


def _sc_top2_body(aff_hbm, out1_hbm, out2_hbm, aff_v0, aff_v1, i1_v, i2_v,
                  sems):
    wid = lax.axis_index("s") * NC + lax.axis_index("c")
    base = wid * TPW
    bufs = [aff_v0, aff_v1]

    lanes = lax.broadcasted_iota(jnp.int32, (L,), 0)

    def make_group(aff_v, off):
        def group(g, carry):
            tok = lanes + g * L
            t1 = jnp.full((L,), -1.0, jnp.float32)
            t2 = jnp.full((L,), -1.0, jnp.float32)
            i1 = jnp.zeros((L,), jnp.int32)
            i2 = jnp.zeros((L,), jnp.int32)
            for e in range(E):
                ev = jnp.full((L,), e, jnp.int32)
                v = plsc.load_gather(aff_v, [tok, ev])
                gt1 = v > t1
                gt2 = v > t2
                t2 = jnp.where(gt1, t1, jnp.where(gt2, v, t2))
                i2 = jnp.where(gt1, i1, jnp.where(gt2, ev, i2))
                t1 = jnp.where(gt1, v, t1)
                i1 = jnp.where(gt1, ev, i1)
            i1_v[pl.ds(off + g * L, L)] = i1
            i2_v[pl.ds(off + g * L, L)] = i2
            return carry
        return group

    pltpu.async_copy(
        aff_hbm.at[pl.ds(base, CHW)], bufs[0], sems.at[0]).start()
    for c in range(NCHK):
        sl = c % 2
        pltpu.make_async_copy(
            aff_hbm.at[pl.ds(base + c * CHW, CHW)], bufs[sl],
            sems.at[sl]).wait()
        if c + 1 < NCHK:
            nsl = (c + 1) % 2
            pltpu.async_copy(
                aff_hbm.at[pl.ds(base + (c + 1) * CHW, CHW)], bufs[nsl],
                sems.at[nsl]).start()
        lax.fori_loop(0, CHW // L, make_group(bufs[sl], c * CHW), 0)

    pltpu.sync_copy(i1_v, out1_hbm.at[pl.ds(base, TPW)])
    pltpu.sync_copy(i2_v, out2_hbm.at[pl.ds(base, TPW)])


@functools.partial(
    pl.kernel,
    out_type=[
        jax.ShapeDtypeStruct((T,), jnp.int32),
        jax.ShapeDtypeStruct((T,), jnp.int32),
    ],
    mesh=plsc.VectorSubcoreMesh(core_axis_name="c", subcore_axis_name="s"),
    scratch_types=[
        pltpu.VMEM((CHW, E), jnp.float32),
        pltpu.VMEM((CHW, E), jnp.float32),
        pltpu.VMEM((TPW,), jnp.int32),
        pltpu.VMEM((TPW,), jnp.int32),
        pltpu.SemaphoreType.DMA((2,)),
    ],
    compiler_params=pltpu.CompilerParams(needs_layout_passes=False),
)
def _sc_top2(aff_hbm, out1_hbm, out2_hbm, aff_v0, aff_v1, i1_v, i2_v, sems):
    _sc_top2_body(aff_hbm, out1_hbm, out2_hbm, aff_v0, aff_v1, i1_v, i2_v,
                  sems)


def kernel(hidden_states, W, b):
    b2 = b.reshape(1, E)
    logits, aff = _tc_logits_softmax(hidden_states, W, b2)
    i1, i2 = _sc_top2(aff)
    idx = jnp.concatenate([i1[:, None], i2[:, None]], axis=1)
    return (logits, aff, idx)


# speedup vs baseline: 1.1231x; 1.0047x over previous
import functools

import jax
import jax.numpy as jnp
from jax import lax
from jax.experimental import pallas as pl
from jax.experimental.pallas import tpu as pltpu
from jax.experimental.pallas import tpu_sc as plsc

T = 32768
H = 768
E = 64
TOP_K = 2
BT = 4096

NC = 2
NS = 16
L = 16
NW = NC * NS
TPW = T // NW


def _router_block(x_ref, w_ref, b_ref, logits_ref, aff_ref):
    x = x_ref[...]
    w = w_ref[...]
    b = b_ref[...]
    logits = jax.lax.dot_general(
        x, w, (((1,), (0,)), ((), ())),
        preferred_element_type=jnp.float32) + b
    logits_ref[...] = logits
    m = jnp.max(logits, axis=1, keepdims=True)
    ex = jnp.exp(logits - m)
    aff_ref[...] = ex / jnp.sum(ex, axis=1, keepdims=True)


def _tc_logits_softmax(hidden_states, W, b2):
    grid = (T // BT,)
    return pl.pallas_call(
        _router_block,
        grid=grid,
        in_specs=[
            pl.BlockSpec((BT, H), lambda i: (i, 0)),
            pl.BlockSpec((H, E), lambda i: (0, 0)),
            pl.BlockSpec((1, E), lambda i: (0, 0)),
        ],
        out_specs=[
            pl.BlockSpec((BT, E), lambda i: (i, 0)),
            pl.BlockSpec((BT, E), lambda i: (i, 0)),
        ],
        out_shape=[
            jax.ShapeDtypeStruct((T, E), jnp.float32),
            jax.ShapeDtypeStruct((T, E), jnp.float32),
        ],
    )(hidden_states, W, b2)


CHW = 256


NCHK = TPW // CHW


def _sc_top2_body(aff_hbm, out1_hbm, out2_hbm, aff_v0, aff_v1, i1_v, i2_v,
                  sems):
    wid = lax.axis_index("s") * NC + lax.axis_index("c")
    base = wid * TPW
    bufs = [aff_v0, aff_v1]

    lanes = lax.broadcasted_iota(jnp.int32, (L,), 0)

    def make_group(aff_v, off):
        def group(g):
            tok = lanes + g * L
            t1 = jnp.full((L,), -1.0, jnp.float32)
            t2 = jnp.full((L,), -1.0, jnp.float32)
            i1 = jnp.zeros((L,), jnp.int32)
            i2 = jnp.zeros((L,), jnp.int32)
            for e in range(E):
                ev = jnp.full((L,), e, jnp.int32)
                v = plsc.load_gather(aff_v, [tok, ev])
                gt1 = v > t1
                gt2 = v > t2
                t2 = jnp.where(gt1, t1, jnp.where(gt2, v, t2))
                i2 = jnp.where(gt1, i1, jnp.where(gt2, ev, i2))
                t1 = jnp.where(gt1, v, t1)
                i1 = jnp.where(gt1, ev, i1)
            i1_v[pl.ds(off + g * L, L)] = i1
            i2_v[pl.ds(off + g * L, L)] = i2
        return group

    pltpu.async_copy(
        aff_hbm.at[pl.ds(base, CHW)], bufs[0], sems.at[0]).start()
    for c in range(NCHK):
        sl = c % 2
        pltpu.make_async_copy(
            aff_hbm.at[pl.ds(base + c * CHW, CHW)], bufs[sl],
            sems.at[sl]).wait()
        if c + 1 < NCHK:
            nsl = (c + 1) % 2
            pltpu.async_copy(
                aff_hbm.at[pl.ds(base + (c + 1) * CHW, CHW)], bufs[nsl],
                sems.at[nsl]).start()
        plsc.parallel_loop(0, CHW // L, unroll=2)(
            make_group(bufs[sl], c * CHW))

    pltpu.sync_copy(i1_v, out1_hbm.at[pl.ds(base, TPW)])
    pltpu.sync_copy(i2_v, out2_hbm.at[pl.ds(base, TPW)])


@functools.partial(
    pl.kernel,
    out_type=[
        jax.ShapeDtypeStruct((T,), jnp.int32),
        jax.ShapeDtypeStruct((T,), jnp.int32),
    ],
    mesh=plsc.VectorSubcoreMesh(core_axis_name="c", subcore_axis_name="s"),
    scratch_types=[
        pltpu.VMEM((CHW, E), jnp.float32),
        pltpu.VMEM((CHW, E), jnp.float32),
        pltpu.VMEM((TPW,), jnp.int32),
        pltpu.VMEM((TPW,), jnp.int32),
        pltpu.SemaphoreType.DMA((2,)),
    ],
    compiler_params=pltpu.CompilerParams(needs_layout_passes=False),
)
def _sc_top2(aff_hbm, out1_hbm, out2_hbm, aff_v0, aff_v1, i1_v, i2_v, sems):
    _sc_top2_body(aff_hbm, out1_hbm, out2_hbm, aff_v0, aff_v1, i1_v, i2_v,
                  sems)


def kernel(hidden_states, W, b):
    b2 = b.reshape(1, E)
    logits, aff = _tc_logits_softmax(hidden_states, W, b2)
    i1, i2 = _sc_top2(aff)
    idx = jnp.concatenate([i1[:, None], i2[:, None]], axis=1)
    return (logits, aff, idx)
